# baseline (speedup 1.0000x reference)
import functools

import jax
import jax.numpy as jnp
import numpy as np
from jax.experimental import pallas as pl


def _window_offsets():
    offs = [(0, 0)]
    for s in range(1, 5):
        for a in (-s, 0, s):
            for b in (-s, 0, s):
                if a != 0 or b != 0:
                    offs.append((a, b))
    return offs


_OFFS = _window_offsets()
_K = len(_OFFS)


def _mask_table(H, W):
    h = np.arange(H)[:, None]
    w = np.arange(W)[None, :]
    rows = []
    for dx, dy in _OFFS:
        valid = (h + dx >= 0) & (h + dx < H) & (w + dy >= 0) & (w + dy < W)
        rows.append(np.where(valid, 0.0, -1e30).reshape(-1))
    return np.stack(rows).astype(np.float32)


def _psla_body(H, W, ft_ref, fte_ref, wf_ref, wg_ref, bf_ref, bg_ref,
               mask_ref, out_ref):
    C = ft_ref.shape[1]
    HW = H * W
    ft = ft_ref[0]
    fte = fte_ref[0]

    et = jnp.dot(wf_ref[...], ft, preferred_element_type=jnp.float32)
    et = et + bf_ref[...]
    ete = jnp.dot(wg_ref[...], fte, preferred_element_type=jnp.float32)
    ete = ete + bg_ref[...]

    pad = 128
    zpad = jnp.zeros((C, pad), jnp.float32)
    etp = jnp.concatenate([zpad, et, zpad], axis=1)

    ones_row = jnp.ones((1, C), jnp.float32)
    rows = []
    for dx, dy in _OFFS:
        d = dx * W + dy
        sh = etp[:, pad + d:pad + d + HW]
        rows.append(jnp.dot(ones_row, ete * sh,
                            precision=jax.lax.Precision.HIGHEST,
                            preferred_element_type=jnp.float32))
    affin = jnp.concatenate(rows, axis=0) + mask_ref[...]

    m = jnp.max(affin, axis=0, keepdims=True)
    e = jnp.exp(affin - m)
    wgt = e / jnp.sum(e, axis=0, keepdims=True)

    s = jnp.dot(ones_row, ft, precision=jax.lax.Precision.HIGHEST,
                preferred_element_type=jnp.float32)
    sp = jnp.concatenate(
        [jnp.zeros((1, pad), jnp.float32), s, jnp.zeros((1, pad), jnp.float32)],
        axis=1)
    outs = []
    for i, (dx, dy) in enumerate(_OFFS):
        d = dx * W + dy
        outs.append(wgt[i:i + 1, :] * sp[:, pad + d:pad + d + HW])
    out_ref[0] = jnp.concatenate(outs, axis=0)


@jax.jit
def kernel(Ft, Ft_epsilon, Wf, bf, Wg, bg):
    B, C, H, W = Ft.shape
    HW = H * W
    ft = Ft.reshape(B, C, HW)
    fte = Ft_epsilon.reshape(B, C, HW)
    mask = jnp.asarray(_mask_table(H, W))

    out = pl.pallas_call(
        functools.partial(_psla_body, H, W),
        grid=(B,),
        in_specs=[
            pl.BlockSpec((1, C, HW), lambda b: (b, 0, 0)),
            pl.BlockSpec((1, C, HW), lambda b: (b, 0, 0)),
            pl.BlockSpec((C, C), lambda b: (0, 0)),
            pl.BlockSpec((C, C), lambda b: (0, 0)),
            pl.BlockSpec((C, 1), lambda b: (0, 0)),
            pl.BlockSpec((C, 1), lambda b: (0, 0)),
            pl.BlockSpec((_K, HW), lambda b: (0, 0)),
        ],
        out_specs=pl.BlockSpec((1, _K, HW), lambda b: (b, 0, 0)),
        out_shape=jax.ShapeDtypeStruct((B, _K, HW), jnp.float32),
    )(ft, fte, Wf, Wg, bf.reshape(C, 1), bg.reshape(C, 1), mask)
    return out.reshape(B, _K, H, W)


# speedup vs baseline: 1.6732x; 1.6732x over previous
import functools

import jax
import jax.numpy as jnp
import numpy as np
from jax.experimental import pallas as pl
from jax.experimental.pallas import tpu as pltpu


def _window_offsets():
    offs = [(0, 0)]
    for s in range(1, 5):
        for a in (-s, 0, s):
            for b in (-s, 0, s):
                if a != 0 or b != 0:
                    offs.append((a, b))
    return offs


_OFFS = _window_offsets()
_K = len(_OFFS)


def _mask_table(H, W):
    h = np.arange(H)[:, None]
    w = np.arange(W)[None, :]
    rows = []
    for dx, dy in _OFFS:
        valid = (h + dx >= 0) & (h + dx < H) & (w + dy >= 0) & (w + dy < W)
        rows.append(np.where(valid, 0.0, -1e30).reshape(-1))
    return np.stack(rows).astype(np.float32)


def _psla_body(H, W, ft_ref, fte_ref, wf_ref, wg_ref, bf_ref, bg_ref,
               mask_ref, out_ref):
    C = ft_ref.shape[1]
    HW = H * W
    pad = 128
    ft = ft_ref[0]
    fte = fte_ref[0]

    et = jnp.dot(wf_ref[...], ft, preferred_element_type=jnp.float32)
    et = et + bf_ref[...]
    ete = jnp.dot(wg_ref[...], fte, preferred_element_type=jnp.float32)
    ete = ete + bg_ref[...]

    zpad = jnp.zeros((C, pad), jnp.float32)
    etp = jnp.concatenate([zpad, et, zpad], axis=1)

    rows = []
    for dx, dy in _OFFS:
        d = dx * W + dy
        sh = etp[:, pad + d:pad + d + HW]
        acc = ete[0:8, :] * sh[0:8, :]
        for r in range(8, C, 8):
            acc = acc + ete[r:r + 8, :] * sh[r:r + 8, :]
        rows.append(jnp.sum(acc, axis=0, keepdims=True))
    affin = jnp.concatenate(rows, axis=0) + mask_ref[...]

    m = jnp.max(affin, axis=0, keepdims=True)
    e = jnp.exp(affin - m)
    wgt = e / jnp.sum(e, axis=0, keepdims=True)

    s = jnp.sum(ft, axis=0, keepdims=True)
    sp = jnp.concatenate(
        [jnp.zeros((1, pad), jnp.float32), s, jnp.zeros((1, pad), jnp.float32)],
        axis=1)
    outs = []
    for i, (dx, dy) in enumerate(_OFFS):
        d = dx * W + dy
        outs.append(wgt[i:i + 1, :] * sp[:, pad + d:pad + d + HW])
    out_ref[0] = jnp.concatenate(outs, axis=0)


@jax.jit
def kernel(Ft, Ft_epsilon, Wf, bf, Wg, bg):
    B, C, H, W = Ft.shape
    HW = H * W
    ft = Ft.reshape(B, C, HW)
    fte = Ft_epsilon.reshape(B, C, HW)
    mask = jnp.asarray(_mask_table(H, W))

    out = pl.pallas_call(
        functools.partial(_psla_body, H, W),
        grid=(B,),
        in_specs=[
            pl.BlockSpec((1, C, HW), lambda b: (b, 0, 0)),
            pl.BlockSpec((1, C, HW), lambda b: (b, 0, 0)),
            pl.BlockSpec((C, C), lambda b: (0, 0)),
            pl.BlockSpec((C, C), lambda b: (0, 0)),
            pl.BlockSpec((C, 1), lambda b: (0, 0)),
            pl.BlockSpec((C, 1), lambda b: (0, 0)),
            pl.BlockSpec((_K, HW), lambda b: (0, 0)),
        ],
        out_specs=pl.BlockSpec((1, _K, HW), lambda b: (b, 0, 0)),
        out_shape=jax.ShapeDtypeStruct((B, _K, HW), jnp.float32),
    )(ft, fte, Wf, Wg, bf.reshape(C, 1), bg.reshape(C, 1), mask)
    return out.reshape(B, _K, H, W)


# speedup vs baseline: 1.6811x; 1.0047x over previous
import functools

import jax
import jax.numpy as jnp
import numpy as np
from jax.experimental import pallas as pl
from jax.experimental.pallas import tpu as pltpu


def _window_offsets():
    offs = [(0, 0)]
    for s in range(1, 5):
        for a in (-s, 0, s):
            for b in (-s, 0, s):
                if a != 0 or b != 0:
                    offs.append((a, b))
    return offs


_OFFS = _window_offsets()
_K = len(_OFFS)


def _mask_table(H, W):
    h = np.arange(H)[:, None]
    w = np.arange(W)[None, :]
    rows = []
    for dx, dy in _OFFS:
        valid = (h + dx >= 0) & (h + dx < H) & (w + dy >= 0) & (w + dy < W)
        rows.append(np.where(valid, 0.0, -1e30).reshape(-1))
    return np.stack(rows).astype(np.float32)


def _psla_body(H, W, ft_ref, fte_ref, wf_ref, wg_ref, bf_ref, bg_ref,
               mask_ref, out_ref):
    for g in range(ft_ref.shape[0]):
        _psla_one(H, W, g, ft_ref, fte_ref, wf_ref, wg_ref, bf_ref, bg_ref,
                  mask_ref, out_ref)


def _psla_one(H, W, g, ft_ref, fte_ref, wf_ref, wg_ref, bf_ref, bg_ref,
              mask_ref, out_ref):
    C = ft_ref.shape[1]
    HW = H * W
    pad = 128
    ft = ft_ref[g]
    fte = fte_ref[g]

    et = jnp.dot(wf_ref[...], ft, preferred_element_type=jnp.float32)
    et = et + bf_ref[...]
    ete = jnp.dot(wg_ref[...], fte, preferred_element_type=jnp.float32)
    ete = ete + bg_ref[...]

    zpad = jnp.zeros((C, pad), jnp.float32)
    etp = jnp.concatenate([zpad, et, zpad], axis=1)

    rows = []
    for dx, dy in _OFFS:
        d = dx * W + dy
        sh = etp[:, pad + d:pad + d + HW]
        acc = ete[0:8, :] * sh[0:8, :]
        for r in range(8, C, 8):
            acc = acc + ete[r:r + 8, :] * sh[r:r + 8, :]
        rows.append(jnp.sum(acc, axis=0, keepdims=True))
    affin = jnp.concatenate(rows, axis=0) + mask_ref[...]

    m = jnp.max(affin, axis=0, keepdims=True)
    e = jnp.exp(affin - m)
    wgt = e / jnp.sum(e, axis=0, keepdims=True)

    s = jnp.sum(ft, axis=0, keepdims=True)
    sp = jnp.concatenate(
        [jnp.zeros((1, pad), jnp.float32), s, jnp.zeros((1, pad), jnp.float32)],
        axis=1)
    outs = []
    for i, (dx, dy) in enumerate(_OFFS):
        d = dx * W + dy
        outs.append(wgt[i:i + 1, :] * sp[:, pad + d:pad + d + HW])
    out_ref[g] = jnp.concatenate(outs, axis=0)


@jax.jit
def kernel(Ft, Ft_epsilon, Wf, bf, Wg, bg):
    B, C, H, W = Ft.shape
    HW = H * W
    ft = Ft.reshape(B, C, HW)
    fte = Ft_epsilon.reshape(B, C, HW)
    mask = jnp.asarray(_mask_table(H, W))

    G = 2
    out = pl.pallas_call(
        functools.partial(_psla_body, H, W),
        grid=(B // G,),
        in_specs=[
            pl.BlockSpec((G, C, HW), lambda b: (b, 0, 0)),
            pl.BlockSpec((G, C, HW), lambda b: (b, 0, 0)),
            pl.BlockSpec((C, C), lambda b: (0, 0)),
            pl.BlockSpec((C, C), lambda b: (0, 0)),
            pl.BlockSpec((C, 1), lambda b: (0, 0)),
            pl.BlockSpec((C, 1), lambda b: (0, 0)),
            pl.BlockSpec((_K, HW), lambda b: (0, 0)),
        ],
        out_specs=pl.BlockSpec((G, _K, HW), lambda b: (b, 0, 0)),
        out_shape=jax.ShapeDtypeStruct((B, _K, HW), jnp.float32),
    )(ft, fte, Wf, Wg, bf.reshape(C, 1), bg.reshape(C, 1), mask)
    return out.reshape(B, _K, H, W)
